# Initial kernel scaffold; baseline (speedup 1.0000x reference)
import jax
import jax.numpy as jnp
from jax.experimental import pallas as pl


def kernel(league_idx, season_idx, home_ts_idx, away_ts_idx, X_home, X_away, mu, gamma_league, hfa_league, delta_season, att, defn, beta_home, beta_away):
    raise NotImplementedError("write your pallas kernel here")


# speedup vs baseline: 3.2386x; 3.2386x over previous
import functools

import jax
import jax.numpy as jnp
from jax import lax
from jax.experimental import pallas as pl
from jax.experimental.pallas import tpu as pltpu
from jax.experimental.pallas import tpu_sc as plsc

N_LEAGUES_PAD = 1024
N_SEASONS_PAD = 64
BATCH = 16384
LANES = 16

_NC = 2
_NS = 16
_NW = _NC * _NS
_CHUNK = BATCH // _NW
_ROWS = _CHUNK // 128


def _sc_gather_kernel(lg_hbm, sn_hbm, hts_hbm, ats_hbm,
                      gam_hbm, hfa_hbm, del_hbm, att_hbm, defn_hbm,
                      bh_hbm, ba_hbm,
                      lg_v, sn_v, hts_v, ats_v,
                      ah_v, aa_v, dh_v, da_v,
                      g_v, h_v, d_v,
                      bh_v, ba_v, sem):
    wid = lax.axis_index("s") * _NC + lax.axis_index("c")
    row0 = wid * _ROWS

    pltpu.sync_copy(lg_hbm.at[pl.ds(row0, _ROWS)], lg_v)
    pltpu.sync_copy(sn_hbm.at[pl.ds(row0, _ROWS)], sn_v)
    pltpu.sync_copy(hts_hbm.at[pl.ds(row0, _ROWS)], hts_v)
    pltpu.sync_copy(ats_hbm.at[pl.ds(row0, _ROWS)], ats_v)

    copies = []
    for j in range(_ROWS):
        copies.append(pltpu.async_copy(att_hbm.at[hts_v.at[j]], ah_v.at[j], sem))
        copies.append(pltpu.async_copy(att_hbm.at[ats_v.at[j]], aa_v.at[j], sem))
        copies.append(pltpu.async_copy(defn_hbm.at[hts_v.at[j]], dh_v.at[j], sem))
        copies.append(pltpu.async_copy(defn_hbm.at[ats_v.at[j]], da_v.at[j], sem))
        copies.append(pltpu.async_copy(gam_hbm.at[lg_v.at[j]], g_v.at[j], sem))
        copies.append(pltpu.async_copy(hfa_hbm.at[lg_v.at[j]], h_v.at[j], sem))
        copies.append(pltpu.async_copy(del_hbm.at[sn_v.at[j]], d_v.at[j], sem))

    for c in copies:
        c.wait()

    for r in range(_ROWS):
        for i in range(128 // LANES):
            s = i * LANES
            gd = g_v[r, pl.ds(s, LANES)] + d_v[r, pl.ds(s, LANES)]
            bh_v[r, pl.ds(s, LANES)] = gd + h_v[r, pl.ds(s, LANES)] + ah_v[r, pl.ds(s, LANES)] - da_v[r, pl.ds(s, LANES)]
            ba_v[r, pl.ds(s, LANES)] = gd + aa_v[r, pl.ds(s, LANES)] - dh_v[r, pl.ds(s, LANES)]

    pltpu.sync_copy(bh_v, bh_hbm.at[pl.ds(row0, _ROWS)])
    pltpu.sync_copy(ba_v, ba_hbm.at[pl.ds(row0, _ROWS)])


def _sc_gather(lg2, sn2, hts2, ats2, gam_p, hfa_p, del_p, att, defn):
    mesh = plsc.VectorSubcoreMesh(core_axis_name="c", subcore_axis_name="s")
    f = pl.kernel(
        _sc_gather_kernel,
        mesh=mesh,
        out_type=[
            jax.ShapeDtypeStruct((BATCH // 128, 128), jnp.float32),
            jax.ShapeDtypeStruct((BATCH // 128, 128), jnp.float32),
        ],
        scratch_types=[
            pltpu.VMEM((_ROWS, 128), jnp.int32),
            pltpu.VMEM((_ROWS, 128), jnp.int32),
            pltpu.VMEM((_ROWS, 128), jnp.int32),
            pltpu.VMEM((_ROWS, 128), jnp.int32),
            pltpu.VMEM((_ROWS, 128), jnp.float32),
            pltpu.VMEM((_ROWS, 128), jnp.float32),
            pltpu.VMEM((_ROWS, 128), jnp.float32),
            pltpu.VMEM((_ROWS, 128), jnp.float32),
            pltpu.VMEM((_ROWS, 128), jnp.float32),
            pltpu.VMEM((_ROWS, 128), jnp.float32),
            pltpu.VMEM((_ROWS, 128), jnp.float32),
            pltpu.VMEM((_ROWS, 128), jnp.float32),
            pltpu.VMEM((_ROWS, 128), jnp.float32),
            pltpu.SemaphoreType.DMA,
        ],
    )
    return f(lg2, sn2, hts2, ats2, gam_p, hfa_p, del_p, att, defn)


def _tc_combine_kernel(mu_ref, bh_ref, ba_ref, xh_ref, xa_ref,
                       beh_ref, bea_ref, oh_ref, oa_ref):
    mu = mu_ref[0, 0]
    lin_h = jnp.sum(xh_ref[...] * beh_ref[...], axis=1)
    lin_a = jnp.sum(xa_ref[...] * bea_ref[...], axis=1)
    log_h = jnp.clip(bh_ref[0, 0, :] + mu + lin_h, -10.0, 10.0)
    log_a = jnp.clip(ba_ref[0, 0, :] + mu + lin_a, -10.0, 10.0)
    oh_ref[0, 0, :] = jnp.exp(log_h)
    oa_ref[0, 0, :] = jnp.exp(log_a)


def _tc_combine(mu, base_h, base_a, X_home, X_away, beta_home, beta_away):
    nblk = 8
    blk = BATCH // nblk
    bh3 = base_h.reshape(nblk, 1, blk)
    ba3 = base_a.reshape(nblk, 1, blk)
    grid = (nblk,)
    out_h, out_a = pl.pallas_call(
        _tc_combine_kernel,
        grid=grid,
        in_specs=[
            pl.BlockSpec((1, 1), lambda i: (0, 0)),
            pl.BlockSpec((1, 1, blk), lambda i: (i, 0, 0)),
            pl.BlockSpec((1, 1, blk), lambda i: (i, 0, 0)),
            pl.BlockSpec((blk, 64), lambda i: (i, 0)),
            pl.BlockSpec((blk, 64), lambda i: (i, 0)),
            pl.BlockSpec((1, 64), lambda i: (0, 0)),
            pl.BlockSpec((1, 64), lambda i: (0, 0)),
        ],
        out_specs=[
            pl.BlockSpec((1, 1, blk), lambda i: (i, 0, 0)),
            pl.BlockSpec((1, 1, blk), lambda i: (i, 0, 0)),
        ],
        out_shape=[
            jax.ShapeDtypeStruct((nblk, 1, blk), jnp.float32),
            jax.ShapeDtypeStruct((nblk, 1, blk), jnp.float32),
        ],
    )(mu.reshape(1, 1), bh3, ba3, X_home, X_away,
      beta_home.reshape(1, 64), beta_away.reshape(1, 64))
    return out_h.reshape(BATCH), out_a.reshape(BATCH)


def kernel(league_idx, season_idx, home_ts_idx, away_ts_idx, X_home, X_away,
           mu, gamma_league, hfa_league, delta_season, att, defn,
           beta_home, beta_away):
    lg2 = league_idx.astype(jnp.int32).reshape(BATCH // 128, 128)
    sn2 = season_idx.astype(jnp.int32).reshape(BATCH // 128, 128)
    hts2 = home_ts_idx.astype(jnp.int32).reshape(BATCH // 128, 128)
    ats2 = away_ts_idx.astype(jnp.int32).reshape(BATCH // 128, 128)
    gam_p = jnp.pad(gamma_league, (0, N_LEAGUES_PAD - gamma_league.shape[0]))
    hfa_p = jnp.pad(hfa_league, (0, N_LEAGUES_PAD - hfa_league.shape[0]))
    del_p = jnp.pad(delta_season, (0, N_SEASONS_PAD - delta_season.shape[0]))

    bh2, ba2 = _sc_gather(lg2, sn2, hts2, ats2, gam_p, hfa_p, del_p, att, defn)
    base_h = bh2.reshape(BATCH)
    base_a = ba2.reshape(BATCH)

    mu_arr = jnp.asarray(mu, jnp.float32)
    return _tc_combine(mu_arr, base_h, base_a, X_home, X_away,
                       beta_home, beta_away)
